# baseline (device time: 11361 ns/iter reference)
import jax
import jax.numpy as jnp
from jax import lax
from jax.experimental import pallas as pl
from jax.experimental.pallas import tpu as pltpu

N_DEV = 8
EPS = 1e-5


def kernel(x, gamma, beta):
    m, n_per = x.shape
    n_global = n_per * N_DEV
    assert m % 128 == 0
    mrows = m // 128

    def body(x_ref, gamma_ref, beta_ref, out_ref,
             stats_ref, recv_ref, send_sems, recv_sems):
        my = lax.axis_index("i")

        barrier_sem = pltpu.get_barrier_semaphore()
        for d in range(1, N_DEV):
            pl.semaphore_signal(
                barrier_sem, inc=1,
                device_id=((my + d) % N_DEV,),
                device_id_type=pl.DeviceIdType.MESH,
            )
        pl.semaphore_wait(barrier_sem, N_DEV - 1)

        xv = x_ref[:, :].astype(jnp.float32)
        stats_ref[0, :, :] = jnp.sum(xv, axis=1).reshape(mrows, 128)
        stats_ref[1, :, :] = jnp.sum(xv * xv, axis=1).reshape(mrows, 128)

        rdmas = []
        for d in range(1, N_DEV):
            rdma = pltpu.make_async_remote_copy(
                src_ref=stats_ref,
                dst_ref=recv_ref.at[d - 1],
                send_sem=send_sems.at[d - 1],
                recv_sem=recv_sems.at[d - 1],
                device_id=((my + d) % N_DEV,),
                device_id_type=pl.DeviceIdType.MESH,
            )
            rdma.start()
            rdmas.append(rdma)
        for rdma in rdmas:
            rdma.wait()

        total = stats_ref[:, :, :]
        for k in range(N_DEV - 1):
            total = total + recv_ref[k, :, :, :]

        inv_n = 1.0 / n_global
        mean_p = total[0] * inv_n
        var_p = total[1] * inv_n - mean_p * mean_p
        rstd_p = lax.rsqrt(var_p + EPS)

        row_id = lax.broadcasted_iota(jnp.int32, (m, 128), 0)
        lane_id = lax.broadcasted_iota(jnp.int32, (m, 128), 1)
        mask = (lane_id == row_id % 128).astype(jnp.float32)
        blk_id = lax.broadcasted_iota(jnp.int32, (m, mrows), 1)
        row_id6 = lax.broadcasted_iota(jnp.int32, (m, mrows), 0)
        bsel = (blk_id == row_id6 // 128).astype(jnp.float32)

        def unpack(p):
            rows = jax.lax.dot(bsel, p, preferred_element_type=jnp.float32)
            return jnp.sum(rows * mask, axis=1, keepdims=True)

        mean = unpack(mean_p)
        rstd = unpack(rstd_p)
        out_ref[:, :] = ((xv - mean) * rstd * gamma_ref[:, :]
                         + beta_ref[:, :]).astype(out_ref.dtype)

    return pl.pallas_call(
        body,
        out_shape=jax.ShapeDtypeStruct((m, n_per), x.dtype),
        in_specs=[
            pl.BlockSpec(memory_space=pltpu.VMEM),
            pl.BlockSpec(memory_space=pltpu.VMEM),
            pl.BlockSpec(memory_space=pltpu.VMEM),
        ],
        out_specs=pl.BlockSpec(memory_space=pltpu.VMEM),
        scratch_shapes=[
            pltpu.VMEM((2, mrows, 128), jnp.float32),
            pltpu.VMEM((N_DEV - 1, 2, mrows, 128), jnp.float32),
            pltpu.SemaphoreType.DMA((N_DEV - 1,)),
            pltpu.SemaphoreType.DMA((N_DEV - 1,)),
        ],
        compiler_params=pltpu.CompilerParams(collective_id=0),
    )(x, gamma.reshape(1, n_per), beta.reshape(1, n_per))


# device time: 11032 ns/iter; 1.0298x vs baseline; 1.0298x over previous
import jax
import jax.numpy as jnp
from jax import lax
from jax.experimental import pallas as pl
from jax.experimental.pallas import tpu as pltpu

N_DEV = 8
EPS = 1e-5


def kernel(x, gamma, beta):
    m, n_per = x.shape
    n_global = n_per * N_DEV
    assert m % 128 == 0
    mrows = m // 128

    def body(x_ref, gamma_ref, beta_ref, out_ref,
             stats_ref, recv_ref, send_sems, recv_sems):
        my = lax.axis_index("i")

        barrier_sem = pltpu.get_barrier_semaphore()
        for d in range(1, N_DEV):
            pl.semaphore_signal(
                barrier_sem, inc=1,
                device_id=((my + d) % N_DEV,),
                device_id_type=pl.DeviceIdType.MESH,
            )

        xv = x_ref[:, :].astype(jnp.float32)
        stats_ref[0, :, :] = jnp.sum(xv, axis=1).reshape(mrows, 128)
        stats_ref[1, :, :] = jnp.sum(xv * xv, axis=1).reshape(mrows, 128)

        pl.semaphore_wait(barrier_sem, N_DEV - 1)

        rdmas = []
        for d in range(1, N_DEV):
            rdma = pltpu.make_async_remote_copy(
                src_ref=stats_ref,
                dst_ref=recv_ref.at[d - 1],
                send_sem=send_sems.at[d - 1],
                recv_sem=recv_sems.at[d - 1],
                device_id=((my + d) % N_DEV,),
                device_id_type=pl.DeviceIdType.MESH,
            )
            rdma.start()
            rdmas.append(rdma)

        row_id = lax.broadcasted_iota(jnp.int32, (m, 128), 0)
        lane_id = lax.broadcasted_iota(jnp.int32, (m, 128), 1)
        mask = (lane_id == row_id % 128).astype(jnp.float32)
        blk_id = lax.broadcasted_iota(jnp.int32, (m, mrows), 1)
        row_id6 = lax.broadcasted_iota(jnp.int32, (m, mrows), 0)
        bsel = (blk_id == row_id6 // 128).astype(jnp.float32)

        for rdma in rdmas:
            rdma.wait()

        total = stats_ref[:, :, :]
        for k in range(N_DEV - 1):
            total = total + recv_ref[k, :, :, :]

        inv_n = 1.0 / n_global
        mean_p = total[0] * inv_n
        var_p = total[1] * inv_n - mean_p * mean_p
        rstd_p = lax.rsqrt(var_p + EPS)

        def unpack(p):
            rows = jax.lax.dot(bsel, p, preferred_element_type=jnp.float32)
            return jnp.sum(rows * mask, axis=1, keepdims=True)

        mean = unpack(mean_p)
        rstd = unpack(rstd_p)
        out_ref[:, :] = ((xv - mean) * rstd * gamma_ref[:, :]
                         + beta_ref[:, :]).astype(out_ref.dtype)

    return pl.pallas_call(
        body,
        out_shape=jax.ShapeDtypeStruct((m, n_per), x.dtype),
        in_specs=[
            pl.BlockSpec(memory_space=pltpu.VMEM),
            pl.BlockSpec(memory_space=pltpu.VMEM),
            pl.BlockSpec(memory_space=pltpu.VMEM),
        ],
        out_specs=pl.BlockSpec(memory_space=pltpu.VMEM),
        scratch_shapes=[
            pltpu.VMEM((2, mrows, 128), jnp.float32),
            pltpu.VMEM((N_DEV - 1, 2, mrows, 128), jnp.float32),
            pltpu.SemaphoreType.DMA((N_DEV - 1,)),
            pltpu.SemaphoreType.DMA((N_DEV - 1,)),
        ],
        compiler_params=pltpu.CompilerParams(collective_id=0),
    )(x, gamma.reshape(1, n_per), beta.reshape(1, n_per))


# device time: 11017 ns/iter; 1.0312x vs baseline; 1.0014x over previous
import jax
import jax.numpy as jnp
from jax import lax
from jax.experimental import pallas as pl
from jax.experimental.pallas import tpu as pltpu

N_DEV = 8
EPS = 1e-5


def kernel(x, gamma, beta):
    m, n_per = x.shape
    n_global = n_per * N_DEV
    assert m % 128 == 0
    mrows = m // 128

    def body(x_ref, gamma_ref, beta_ref, out_ref,
             stats_ref, recv_ref, send_sems, recv_sems):
        my = lax.axis_index("i")

        barrier_sem = pltpu.get_barrier_semaphore()
        for d in range(1, N_DEV):
            pl.semaphore_signal(
                barrier_sem, inc=1,
                device_id=((my + d) % N_DEV,),
                device_id_type=pl.DeviceIdType.MESH,
            )

        xv = x_ref[:, :].astype(jnp.float32)
        stats_ref[0, :, :] = jnp.sum(xv, axis=1).reshape(mrows, 128)
        stats_ref[1, :, :] = jnp.sum(xv * xv, axis=1).reshape(mrows, 128)

        pl.semaphore_wait(barrier_sem, N_DEV - 1)

        rdmas = []
        for d in range(1, N_DEV):
            rdma = pltpu.make_async_remote_copy(
                src_ref=stats_ref,
                dst_ref=recv_ref.at[d - 1],
                send_sem=send_sems.at[d - 1],
                recv_sem=recv_sems.at[d - 1],
                device_id=((my + d) % N_DEV,),
                device_id_type=pl.DeviceIdType.MESH,
            )
            rdma.start()
            rdmas.append(rdma)

        row_id = lax.broadcasted_iota(jnp.int32, (m, 128), 0)
        lane_id = lax.broadcasted_iota(jnp.int32, (m, 128), 1)
        mask = (lane_id == row_id % 128).astype(jnp.float32)
        blk_id = lax.broadcasted_iota(jnp.int32, (m, mrows), 1)
        row_id6 = lax.broadcasted_iota(jnp.int32, (m, mrows), 0)
        bsel = (blk_id == row_id6 // 128).astype(jnp.float32)

        for rdma in rdmas:
            rdma.wait()

        total = stats_ref[:, :, :]
        for k in range(N_DEV - 1):
            total = total + recv_ref[k, :, :, :]

        inv_n = 1.0 / n_global
        mean_p = total[0] * inv_n
        var_p = total[1] * inv_n - mean_p * mean_p
        rstd_p = lax.rsqrt(var_p + EPS)

        def unpack(p):
            rows = jax.lax.dot(bsel, p, preferred_element_type=jnp.float32)
            return jnp.sum(rows * mask, axis=1, keepdims=True)

        mean = unpack(mean_p)
        rstd = unpack(rstd_p)
        xb = x_ref[:, :].astype(jnp.bfloat16)
        meanb = mean.astype(jnp.bfloat16)
        rstdb = rstd.astype(jnp.bfloat16)
        gammab = gamma_ref[:, :].astype(jnp.bfloat16)
        betab = beta_ref[:, :].astype(jnp.bfloat16)
        out_ref[:, :] = ((xb - meanb) * rstdb * gammab
                         + betab).astype(out_ref.dtype)

    return pl.pallas_call(
        body,
        out_shape=jax.ShapeDtypeStruct((m, n_per), x.dtype),
        in_specs=[
            pl.BlockSpec(memory_space=pltpu.VMEM),
            pl.BlockSpec(memory_space=pltpu.VMEM),
            pl.BlockSpec(memory_space=pltpu.VMEM),
        ],
        out_specs=pl.BlockSpec(memory_space=pltpu.VMEM),
        scratch_shapes=[
            pltpu.VMEM((2, mrows, 128), jnp.float32),
            pltpu.VMEM((N_DEV - 1, 2, mrows, 128), jnp.float32),
            pltpu.SemaphoreType.DMA((N_DEV - 1,)),
            pltpu.SemaphoreType.DMA((N_DEV - 1,)),
        ],
        compiler_params=pltpu.CompilerParams(collective_id=0),
    )(x, gamma.reshape(1, n_per), beta.reshape(1, n_per))
